# baseline (speedup 1.0000x reference)
import functools
import math

import jax
import jax.numpy as jnp
from jax import lax
from jax.experimental import pallas as pl
from jax.experimental.pallas import tpu as pltpu
from jax.experimental.pallas import tpu_sc as plsc

D_MODEL = 512
SCALE = math.sqrt(D_MODEL)
LANES = 16

NUM_CORES = 2
NUM_SUBCORES = 16
NW = NUM_CORES * NUM_SUBCORES

CHUNK = 64


def _emb_body(nchunks, table_hbm, idx_hbm, out_hbm, idx_v, rows_v, sem_in):
    wid = lax.axis_index("s") * NUM_CORES + lax.axis_index("c")
    b_per_w = nchunks * CHUNK
    base = wid * b_per_w

    pltpu.sync_copy(idx_hbm.at[wid], idx_v)

    def chunk_body(c, _):
        pltpu.async_copy(table_hbm.at[idx_v.at[c]], rows_v, sem_in).wait()

        def row_body(r, _):
            for j in range(D_MODEL // LANES):
                sl = pl.ds(j * LANES, LANES)
                rows_v[r, sl] = rows_v[r, sl] * SCALE
            return 0

        lax.fori_loop(0, CHUNK, row_body, 0)

        pltpu.sync_copy(rows_v, out_hbm.at[pl.ds(base + c * CHUNK, CHUNK)])
        return 0

    lax.fori_loop(0, nchunks, chunk_body, 0)


@functools.lru_cache(maxsize=None)
def _make_emb(B):
    assert B % (NW * CHUNK) == 0
    nchunks = B // (NW * CHUNK)
    mesh = plsc.VectorSubcoreMesh(
        core_axis_name="c", subcore_axis_name="s",
        num_cores=NUM_CORES, num_subcores=NUM_SUBCORES)
    return pl.kernel(
        functools.partial(_emb_body, nchunks),
        out_type=jax.ShapeDtypeStruct((B, D_MODEL), jnp.float32),
        mesh=mesh,
        scratch_types=[
            pltpu.VMEM((nchunks, CHUNK), jnp.int32),
            pltpu.VMEM((CHUNK, D_MODEL), jnp.float32),
            pltpu.SemaphoreType.DMA,
        ],
    )


def kernel(x, table):
    orig_shape = x.shape
    B = x.size
    idx = x.reshape(NW, B // (NW * CHUNK), CHUNK).astype(jnp.int32)
    out = _make_emb(B)(table, idx)
    return out.reshape(*orig_shape, D_MODEL)


# speedup vs baseline: 1.1787x; 1.1787x over previous
import functools
import math

import jax
import jax.numpy as jnp
from jax import lax
from jax.experimental import pallas as pl
from jax.experimental.pallas import tpu as pltpu
from jax.experimental.pallas import tpu_sc as plsc

D_MODEL = 512
SCALE = math.sqrt(D_MODEL)
LANES = 16

NUM_CORES = 2
NUM_SUBCORES = 16
NW = NUM_CORES * NUM_SUBCORES

CHUNK = 80
NBUF = 2


def _emb_body(nchunks, table_hbm, idx_hbm, out_hbm,
              idx_v, bufs, gsems, ssems):
    wid = lax.axis_index("s") * NUM_CORES + lax.axis_index("c")
    b_per_w = nchunks * CHUNK
    base = wid * b_per_w

    def out_at(c):
        return out_hbm.at[pl.ds(base + c * CHUNK, CHUNK)]

    pltpu.sync_copy(idx_hbm.at[wid], idx_v)

    pltpu.async_copy(table_hbm.at[idx_v.at[0]], bufs[0], gsems[0])

    def pair_body(c0, _):
        for b in range(NBUF):
            c = c0 + b
            b2 = (b + 1) % NBUF
            pltpu.make_async_copy(
                table_hbm.at[pl.ds(0, CHUNK)], bufs[b], gsems[b]).wait()

            @pl.when(c + 1 < nchunks)
            def _():
                @pl.when(c >= 1)
                def _():
                    pltpu.make_async_copy(bufs[b2], out_at(c - 1),
                                          ssems[b2]).wait()
                pltpu.async_copy(
                    table_hbm.at[idx_v.at[c + 1]], bufs[b2], gsems[b2])

            def row_body(r, _):
                for j in range(D_MODEL // LANES):
                    sl = pl.ds(j * LANES, LANES)
                    bufs[b][r, sl] = bufs[b][r, sl] * SCALE
                return 0

            lax.fori_loop(0, CHUNK, row_body, 0)

            pltpu.async_copy(bufs[b], out_at(c), ssems[b])
        return 0

    lax.fori_loop(0, nchunks // NBUF, lambda i, a: pair_body(i * NBUF, a), 0)

    for b in range(NBUF):
        c = nchunks - NBUF + b
        pltpu.make_async_copy(bufs[b % NBUF], out_at(c),
                              ssems[c % NBUF]).wait()


@functools.lru_cache(maxsize=None)
def _make_emb(B):
    assert B % (NW * CHUNK) == 0
    nchunks = B // (NW * CHUNK)
    assert nchunks % NBUF == 0
    mesh = plsc.VectorSubcoreMesh(
        core_axis_name="c", subcore_axis_name="s",
        num_cores=NUM_CORES, num_subcores=NUM_SUBCORES)
    return pl.kernel(
        functools.partial(_emb_body, nchunks),
        out_type=jax.ShapeDtypeStruct((B, D_MODEL), jnp.float32),
        mesh=mesh,
        scratch_types=[
            pltpu.VMEM((nchunks, CHUNK), jnp.int32),
            [pltpu.VMEM((CHUNK, D_MODEL), jnp.float32) for _ in range(NBUF)],
            [pltpu.SemaphoreType.DMA for _ in range(NBUF)],
            [pltpu.SemaphoreType.DMA for _ in range(NBUF)],
        ],
    )


def kernel(x, table):
    orig_shape = x.shape
    B = x.size
    idx = x.reshape(NW, B // (NW * CHUNK), CHUNK).astype(jnp.int32)
    out = _make_emb(B)(table, idx)
    return out.reshape(*orig_shape, D_MODEL)


# speedup vs baseline: 1.1798x; 1.0009x over previous
import functools
import math

import jax
import jax.numpy as jnp
from jax import lax
from jax.experimental import pallas as pl
from jax.experimental.pallas import tpu as pltpu
from jax.experimental.pallas import tpu_sc as plsc

D_MODEL = 512
SCALE = math.sqrt(D_MODEL)
LANES = 16

NUM_CORES = 2
NUM_SUBCORES = 16
NW = NUM_CORES * NUM_SUBCORES

CHUNK = 80
NBUF = 2


def _emb_body(nchunks, table_hbm, idx_hbm, out_hbm,
              idx_v, bufs, gsems, ssems):
    wid = lax.axis_index("s") * NUM_CORES + lax.axis_index("c")
    b_per_w = nchunks * CHUNK
    base = wid * b_per_w

    def out_at(c):
        return out_hbm.at[pl.ds(base + c * CHUNK, CHUNK)]

    pltpu.sync_copy(idx_hbm.at[wid], idx_v)

    pltpu.async_copy(table_hbm.at[idx_v.at[0]], bufs[0], gsems[0])

    def pair_body(c0, _):
        for b in range(NBUF):
            c = c0 + b
            b2 = (b + 1) % NBUF
            pltpu.make_async_copy(
                table_hbm.at[pl.ds(0, CHUNK)], bufs[b], gsems[b]).wait()

            @pl.when(c + 1 < nchunks)
            def _():
                @pl.when(c >= 1)
                def _():
                    pltpu.make_async_copy(bufs[b2], out_at(c - 1),
                                          ssems[b2]).wait()
                pltpu.async_copy(
                    table_hbm.at[idx_v.at[c + 1]], bufs[b2], gsems[b2])

            @plsc.parallel_loop(0, CHUNK, step=1, unroll=2)
            def _(r):
                for j in range(D_MODEL // LANES):
                    sl = pl.ds(j * LANES, LANES)
                    bufs[b][r, sl] = bufs[b][r, sl] * SCALE

            pltpu.async_copy(bufs[b], out_at(c), ssems[b])
        return 0

    lax.fori_loop(0, nchunks // NBUF, lambda i, a: pair_body(i * NBUF, a), 0)

    for b in range(NBUF):
        c = nchunks - NBUF + b
        pltpu.make_async_copy(bufs[b % NBUF], out_at(c),
                              ssems[c % NBUF]).wait()


@functools.lru_cache(maxsize=None)
def _make_emb(B):
    assert B % (NW * CHUNK) == 0
    nchunks = B // (NW * CHUNK)
    assert nchunks % NBUF == 0
    mesh = plsc.VectorSubcoreMesh(
        core_axis_name="c", subcore_axis_name="s",
        num_cores=NUM_CORES, num_subcores=NUM_SUBCORES)
    return pl.kernel(
        functools.partial(_emb_body, nchunks),
        out_type=jax.ShapeDtypeStruct((B, D_MODEL), jnp.float32),
        mesh=mesh,
        scratch_types=[
            pltpu.VMEM((nchunks, CHUNK), jnp.int32),
            [pltpu.VMEM((CHUNK, D_MODEL), jnp.float32) for _ in range(NBUF)],
            [pltpu.SemaphoreType.DMA for _ in range(NBUF)],
            [pltpu.SemaphoreType.DMA for _ in range(NBUF)],
        ],
    )


def kernel(x, table):
    orig_shape = x.shape
    B = x.size
    idx = x.reshape(NW, B // (NW * CHUNK), CHUNK).astype(jnp.int32)
    out = _make_emb(B)(table, idx)
    return out.reshape(*orig_shape, D_MODEL)
